# Initial kernel scaffold; baseline (speedup 1.0000x reference)
import jax
import jax.numpy as jnp
from jax.experimental import pallas as pl


def kernel(features, edge_index, W_self0, b_self0, W_neigh0, W_self1, b_self1, W_neigh1, W_lin, b_lin, g0, be0, g1, be1):
    raise NotImplementedError("write your pallas kernel here")


# speedup vs baseline: 4.6841x; 4.6841x over previous
import jax
import jax.numpy as jnp
from jax import lax
from jax.experimental import pallas as pl
from jax.experimental.pallas import tpu as pltpu
from jax.experimental.pallas import tpu_sc as plsc

N = 10000
E = 320000
D = 128
C = 64

NC = 2
NS = 16
NW = NC * NS
EW = E // NW
CHUNK = 80
NCHUNK = EW // CHUNK
SUB_STRIDE = 640
ZCH = 80

_MESH = plsc.VectorSubcoreMesh(core_axis_name="core",
                               subcore_axis_name="subcore",
                               num_cores=NC, num_subcores=NS)


def _fill_vmem_2d(ref, nrows, ncols, value):
    @pl.loop(0, nrows)
    def _(i):
        @pl.loop(0, ncols, step=16)
        def _(j):
            ref.at[pl.ds(i, 1), pl.ds(j, 16)][...] = jnp.full(
                (1, 16), value, jnp.float32)


def _per_sub_chunks(s, fn):
    for k in range(SUB_STRIDE // ZCH):
        off = s * SUB_STRIDE + k * ZCH

        @pl.when(off < N)
        def _():
            fn(off)


def _sc_agg_body(h_hbm, src_hbm, dst_hbm, agg_out, src_v, dst_v, rows_v,
                 zrow_v, agg_sh):
    c = lax.axis_index("core")
    s = lax.axis_index("subcore")
    gbase = (c * NS + s) * EW

    _fill_vmem_2d(zrow_v, ZCH, D, 0.0)
    _per_sub_chunks(
        s, lambda off: pltpu.sync_copy(zrow_v, agg_sh.at[pl.ds(off, ZCH)]))
    plsc.subcore_barrier()

    @pl.loop(0, NCHUNK)
    def _(i):
        off = gbase + i * CHUNK
        pltpu.sync_copy(src_hbm.at[pl.ds(off, CHUNK)], src_v)
        pltpu.sync_copy(dst_hbm.at[pl.ds(off, CHUNK)], dst_v)
        pltpu.sync_copy(h_hbm.at[src_v], rows_v)
        pltpu.sync_copy(rows_v, agg_sh.at[dst_v], add=True)
    plsc.subcore_barrier()

    _per_sub_chunks(
        s, lambda off: pltpu.sync_copy(agg_sh.at[pl.ds(off, ZCH)],
                                       agg_out.at[c, pl.ds(off, ZCH)]))


_sc_agg = pl.kernel(
    _sc_agg_body,
    out_type=(jax.ShapeDtypeStruct((NC, N, D), jnp.float32),),
    mesh=_MESH,
    scratch_types=[
        pltpu.VMEM((CHUNK,), jnp.int32),
        pltpu.VMEM((CHUNK,), jnp.int32),
        pltpu.VMEM((CHUNK, D), jnp.float32),
        pltpu.VMEM((ZCH, D), jnp.float32),
        pltpu.VMEM_SHARED((N, D), jnp.float32),
    ])


def _sc_deg_body(dst_hbm, deg_out, dst_v, ones_v, zrow_v, deg_sh):
    c = lax.axis_index("core")
    s = lax.axis_index("subcore")
    gbase = (c * NS + s) * EW

    _fill_vmem_2d(zrow_v, ZCH, D, 0.0)
    _per_sub_chunks(
        s, lambda off: pltpu.sync_copy(zrow_v, deg_sh.at[pl.ds(off, ZCH)]))
    _fill_vmem_2d(ones_v, CHUNK, D, 1.0)
    plsc.subcore_barrier()

    @pl.loop(0, NCHUNK)
    def _(i):
        off = gbase + i * CHUNK
        pltpu.sync_copy(dst_hbm.at[pl.ds(off, CHUNK)], dst_v)
        pltpu.sync_copy(ones_v, deg_sh.at[dst_v], add=True)
    plsc.subcore_barrier()

    _per_sub_chunks(
        s, lambda off: pltpu.sync_copy(deg_sh.at[pl.ds(off, ZCH)],
                                       deg_out.at[c, pl.ds(off, ZCH)]))


_sc_deg = pl.kernel(
    _sc_deg_body,
    out_type=(jax.ShapeDtypeStruct((NC, N, D), jnp.float32),),
    mesh=_MESH,
    scratch_types=[
        pltpu.VMEM((CHUNK,), jnp.int32),
        pltpu.VMEM((CHUNK, D), jnp.float32),
        pltpu.VMEM((ZCH, D), jnp.float32),
        pltpu.VMEM_SHARED((N, D), jnp.float32),
    ])


BR = 1000


def _tc_layer0_body(h, a0, a1, d0, d1, ws, wn, b, g, be, o):
    deg = jnp.maximum(d0[:, :1] + d1[:, :1], 1.0)
    agg = (a0[...] + a1[...]) / deg
    z = (jnp.dot(h[...], ws[...], preferred_element_type=jnp.float32)
         + jnp.dot(agg, wn[...], preferred_element_type=jnp.float32)
         + b[...])
    mu = jnp.mean(z, axis=-1, keepdims=True)
    var = jnp.mean((z - mu) ** 2, axis=-1, keepdims=True)
    y = (z - mu) / jnp.sqrt(var + 1e-5) * g[...] + be[...]
    o[...] = jnp.maximum(y, 0.0)


def _tc_layer1_body(h, a0, a1, d0, d1, ws, wn, b, g, be, wl, bl, o):
    deg = jnp.maximum(d0[:, :1] + d1[:, :1], 1.0)
    agg = (a0[...] + a1[...]) / deg
    z = (jnp.dot(h[...], ws[...], preferred_element_type=jnp.float32)
         + jnp.dot(agg, wn[...], preferred_element_type=jnp.float32)
         + b[...])
    mu = jnp.mean(z, axis=-1, keepdims=True)
    var = jnp.mean((z - mu) ** 2, axis=-1, keepdims=True)
    y = (z - mu) / jnp.sqrt(var + 1e-5) * g[...] + be[...]
    y = jnp.maximum(y, 0.0)
    o[...] = jnp.dot(y, wl[...], preferred_element_type=jnp.float32) + bl[...]


def _row_spec(w):
    return pl.BlockSpec((BR, w), lambda i: (i, 0))


def _full_spec(r, c_):
    return pl.BlockSpec((r, c_), lambda i: (0, 0))


_tc_layer0 = pl.pallas_call(
    _tc_layer0_body,
    grid=(N // BR,),
    in_specs=[_row_spec(D), _row_spec(D), _row_spec(D),
              _row_spec(D), _row_spec(D),
              _full_spec(D, D), _full_spec(D, D),
              _full_spec(1, D), _full_spec(1, D), _full_spec(1, D)],
    out_specs=_row_spec(D),
    out_shape=jax.ShapeDtypeStruct((N, D), jnp.float32),
)

_tc_layer1 = pl.pallas_call(
    _tc_layer1_body,
    grid=(N // BR,),
    in_specs=[_row_spec(D), _row_spec(D), _row_spec(D),
              _row_spec(D), _row_spec(D),
              _full_spec(D, D), _full_spec(D, D),
              _full_spec(1, D), _full_spec(1, D), _full_spec(1, D),
              _full_spec(D, C), _full_spec(1, C)],
    out_specs=pl.BlockSpec((BR, C), lambda i: (i, 0)),
    out_shape=jax.ShapeDtypeStruct((N, C), jnp.float32),
)


def kernel(features, edge_index, W_self0, b_self0, W_neigh0, W_self1,
           b_self1, W_neigh1, W_lin, b_lin, g0, be0, g1, be1):
    src, dst = edge_index[0], edge_index[1]
    (deg_p,) = _sc_deg(dst)
    (agg_p,) = _sc_agg(features, src, dst)
    h1 = _tc_layer0(features, agg_p[0], agg_p[1], deg_p[0], deg_p[1],
                    W_self0, W_neigh0, b_self0.reshape(1, D),
                    g0.reshape(1, D), be0.reshape(1, D))
    (agg_p1,) = _sc_agg(h1, src, dst)
    out = _tc_layer1(h1, agg_p1[0], agg_p1[1], deg_p[0], deg_p[1],
                     W_self1, W_neigh1, b_self1.reshape(1, D),
                     g1.reshape(1, D), be1.reshape(1, D),
                     W_lin, b_lin.reshape(1, C))
    return out


# speedup vs baseline: 7.4244x; 1.5850x over previous
import jax
import jax.numpy as jnp
from jax import lax
from jax.experimental import pallas as pl
from jax.experimental.pallas import tpu as pltpu
from jax.experimental.pallas import tpu_sc as plsc

N = 10000
E = 320000
D = 128
C = 64

NC = 2
NS = 16
NW = NC * NS
EW = E // NW
CHUNK = 80
NCHUNK = EW // CHUNK
SUB_STRIDE = 640
ZCH = 80

_MESH = plsc.VectorSubcoreMesh(core_axis_name="core",
                               subcore_axis_name="subcore",
                               num_cores=NC, num_subcores=NS)


def _fill_vmem_2d(ref, nrows, ncols, value):
    @pl.loop(0, nrows)
    def _(i):
        @pl.loop(0, ncols, step=16)
        def _(j):
            ref.at[pl.ds(i, 1), pl.ds(j, 16)][...] = jnp.full(
                (1, 16), value, jnp.float32)


def _per_sub_chunks(s, fn):
    for k in range(SUB_STRIDE // ZCH):
        off = s * SUB_STRIDE + k * ZCH

        @pl.when(off < N)
        def _():
            fn(off)


def _sc_agg_body(h_hbm, src_hbm, dst_hbm, agg_out, src_a, dst_a, src_b,
                 dst_b, rows_a, rows_b, zrow_v, agg_sh, gsem_a, gsem_b):
    c = lax.axis_index("core")
    s = lax.axis_index("subcore")
    gbase = (c * NS + s) * EW

    _fill_vmem_2d(zrow_v, ZCH, D, 0.0)
    _per_sub_chunks(
        s, lambda off: pltpu.sync_copy(zrow_v, agg_sh.at[pl.ds(off, ZCH)]))
    plsc.subcore_barrier()

    def load_and_gather(i, sv, dv, rv, sem):
        off = gbase + i * CHUNK
        pltpu.sync_copy(src_hbm.at[pl.ds(off, CHUNK)], sv)
        pltpu.sync_copy(dst_hbm.at[pl.ds(off, CHUNK)], dv)
        pltpu.async_copy(h_hbm.at[sv], rv, sem)

    def finish_and_scatter(sv, dv, rv, sem):
        pltpu.make_async_copy(h_hbm.at[sv], rv, sem).wait()
        pltpu.sync_copy(rv, agg_sh.at[dv], add=True)

    load_and_gather(0, src_a, dst_a, rows_a, gsem_a)

    @pl.loop(0, NCHUNK - 1, step=2)
    def _(i):
        load_and_gather(i + 1, src_b, dst_b, rows_b, gsem_b)
        finish_and_scatter(src_a, dst_a, rows_a, gsem_a)
        load_and_gather(i + 2, src_a, dst_a, rows_a, gsem_a)
        finish_and_scatter(src_b, dst_b, rows_b, gsem_b)

    finish_and_scatter(src_a, dst_a, rows_a, gsem_a)
    plsc.subcore_barrier()

    _per_sub_chunks(
        s, lambda off: pltpu.sync_copy(agg_sh.at[pl.ds(off, ZCH)],
                                       agg_out.at[c, pl.ds(off, ZCH)]))


_sc_agg = pl.kernel(
    _sc_agg_body,
    out_type=(jax.ShapeDtypeStruct((NC, N, D), jnp.float32),),
    mesh=_MESH,
    scratch_types=[
        pltpu.VMEM((CHUNK,), jnp.int32),
        pltpu.VMEM((CHUNK,), jnp.int32),
        pltpu.VMEM((CHUNK,), jnp.int32),
        pltpu.VMEM((CHUNK,), jnp.int32),
        pltpu.VMEM((CHUNK, D), jnp.float32),
        pltpu.VMEM((CHUNK, D), jnp.float32),
        pltpu.VMEM((ZCH, D), jnp.float32),
        pltpu.VMEM_SHARED((N, D), jnp.float32),
        pltpu.SemaphoreType.DMA,
        pltpu.SemaphoreType.DMA,
    ])


def _sc_deg_body(dst_hbm, deg_out, dst_a, dst_b, ones_v, zrow_v, deg_sh,
                 isem_a, isem_b):
    c = lax.axis_index("core")
    s = lax.axis_index("subcore")
    gbase = (c * NS + s) * EW

    _fill_vmem_2d(zrow_v, ZCH, D, 0.0)
    _per_sub_chunks(
        s, lambda off: pltpu.sync_copy(zrow_v, deg_sh.at[pl.ds(off, ZCH)]))
    _fill_vmem_2d(ones_v, CHUNK, D, 1.0)
    plsc.subcore_barrier()

    def load(i, dv, sem):
        off = gbase + i * CHUNK
        pltpu.async_copy(dst_hbm.at[pl.ds(off, CHUNK)], dv, sem)

    def scatter(dv, sem):
        pltpu.make_async_copy(dst_hbm.at[pl.ds(0, CHUNK)], dv, sem).wait()
        pltpu.sync_copy(ones_v, deg_sh.at[dv], add=True)

    load(0, dst_a, isem_a)

    @pl.loop(0, NCHUNK - 1, step=2)
    def _(i):
        load(i + 1, dst_b, isem_b)
        scatter(dst_a, isem_a)
        load(i + 2, dst_a, isem_a)
        scatter(dst_b, isem_b)

    scatter(dst_a, isem_a)
    plsc.subcore_barrier()

    _per_sub_chunks(
        s, lambda off: pltpu.sync_copy(deg_sh.at[pl.ds(off, ZCH)],
                                       deg_out.at[c, pl.ds(off, ZCH)]))


_sc_deg = pl.kernel(
    _sc_deg_body,
    out_type=(jax.ShapeDtypeStruct((NC, N, D), jnp.float32),),
    mesh=_MESH,
    scratch_types=[
        pltpu.VMEM((CHUNK,), jnp.int32),
        pltpu.VMEM((CHUNK,), jnp.int32),
        pltpu.VMEM((CHUNK, D), jnp.float32),
        pltpu.VMEM((ZCH, D), jnp.float32),
        pltpu.VMEM_SHARED((N, D), jnp.float32),
        pltpu.SemaphoreType.DMA,
        pltpu.SemaphoreType.DMA,
    ])


BR = 1000


def _tc_layer0_body(h, a0, a1, d0, d1, ws, wn, b, g, be, o):
    deg = jnp.maximum(d0[:, :1] + d1[:, :1], 1.0)
    agg = (a0[...] + a1[...]) / deg
    z = (jnp.dot(h[...], ws[...], preferred_element_type=jnp.float32)
         + jnp.dot(agg, wn[...], preferred_element_type=jnp.float32)
         + b[...])
    mu = jnp.mean(z, axis=-1, keepdims=True)
    var = jnp.mean((z - mu) ** 2, axis=-1, keepdims=True)
    y = (z - mu) / jnp.sqrt(var + 1e-5) * g[...] + be[...]
    o[...] = jnp.maximum(y, 0.0)


def _tc_layer1_body(h, a0, a1, d0, d1, ws, wn, b, g, be, wl, bl, o):
    deg = jnp.maximum(d0[:, :1] + d1[:, :1], 1.0)
    agg = (a0[...] + a1[...]) / deg
    z = (jnp.dot(h[...], ws[...], preferred_element_type=jnp.float32)
         + jnp.dot(agg, wn[...], preferred_element_type=jnp.float32)
         + b[...])
    mu = jnp.mean(z, axis=-1, keepdims=True)
    var = jnp.mean((z - mu) ** 2, axis=-1, keepdims=True)
    y = (z - mu) / jnp.sqrt(var + 1e-5) * g[...] + be[...]
    y = jnp.maximum(y, 0.0)
    o[...] = jnp.dot(y, wl[...], preferred_element_type=jnp.float32) + bl[...]


def _row_spec(w):
    return pl.BlockSpec((BR, w), lambda i: (i, 0))


def _full_spec(r, c_):
    return pl.BlockSpec((r, c_), lambda i: (0, 0))


_tc_layer0 = pl.pallas_call(
    _tc_layer0_body,
    grid=(N // BR,),
    in_specs=[_row_spec(D), _row_spec(D), _row_spec(D),
              _row_spec(D), _row_spec(D),
              _full_spec(D, D), _full_spec(D, D),
              _full_spec(1, D), _full_spec(1, D), _full_spec(1, D)],
    out_specs=_row_spec(D),
    out_shape=jax.ShapeDtypeStruct((N, D), jnp.float32),
)

_tc_layer1 = pl.pallas_call(
    _tc_layer1_body,
    grid=(N // BR,),
    in_specs=[_row_spec(D), _row_spec(D), _row_spec(D),
              _row_spec(D), _row_spec(D),
              _full_spec(D, D), _full_spec(D, D),
              _full_spec(1, D), _full_spec(1, D), _full_spec(1, D),
              _full_spec(D, C), _full_spec(1, C)],
    out_specs=pl.BlockSpec((BR, C), lambda i: (i, 0)),
    out_shape=jax.ShapeDtypeStruct((N, C), jnp.float32),
)


def kernel(features, edge_index, W_self0, b_self0, W_neigh0, W_self1,
           b_self1, W_neigh1, W_lin, b_lin, g0, be0, g1, be1):
    src, dst = edge_index[0], edge_index[1]
    (deg_p,) = _sc_deg(dst)
    (agg_p,) = _sc_agg(features, src, dst)
    h1 = _tc_layer0(features, agg_p[0], agg_p[1], deg_p[0], deg_p[1],
                    W_self0, W_neigh0, b_self0.reshape(1, D),
                    g0.reshape(1, D), be0.reshape(1, D))
    (agg_p1,) = _sc_agg(h1, src, dst)
    out = _tc_layer1(h1, agg_p1[0], agg_p1[1], deg_p[0], deg_p[1],
                     W_self1, W_neigh1, b_self1.reshape(1, D),
                     g1.reshape(1, D), be1.reshape(1, D),
                     W_lin, b_lin.reshape(1, C))
    return out


# speedup vs baseline: 8.6031x; 1.1588x over previous
import jax
import jax.numpy as jnp
from jax import lax
from jax.experimental import pallas as pl
from jax.experimental.pallas import tpu as pltpu
from jax.experimental.pallas import tpu_sc as plsc

N = 10000
E = 320000
D = 128
C = 64

NC = 2
NS = 16
NW = NC * NS
EW = E // NW
CHUNK = 128
NFULL = EW // CHUNK
TAIL = EW - NFULL * CHUNK
SUB_STRIDE = 640
ZCH = 80

_MESH = plsc.VectorSubcoreMesh(core_axis_name="core",
                               subcore_axis_name="subcore",
                               num_cores=NC, num_subcores=NS)


def _fill_vmem_2d(ref, nrows, ncols, value):
    @pl.loop(0, nrows)
    def _(i):
        @pl.loop(0, ncols, step=16)
        def _(j):
            ref.at[pl.ds(i, 1), pl.ds(j, 16)][...] = jnp.full(
                (1, 16), value, jnp.float32)


def _per_sub_chunks(s, fn):
    for k in range(SUB_STRIDE // ZCH):
        off = s * SUB_STRIDE + k * ZCH

        @pl.when(off < N)
        def _():
            fn(off)


def _sc_agg_body(h_hbm, src_hbm, dst_hbm, agg_out, src_a, dst_a, src_b,
                 dst_b, src_t, dst_t, rows_a, rows_b, rows_t, zrow_v,
                 agg_sh, gsem_a, gsem_b):
    c = lax.axis_index("core")
    s = lax.axis_index("subcore")
    gbase = (c * NS + s) * EW

    _fill_vmem_2d(zrow_v, ZCH, D, 0.0)
    _per_sub_chunks(
        s, lambda off: pltpu.sync_copy(zrow_v, agg_sh.at[pl.ds(off, ZCH)]))
    plsc.subcore_barrier()

    def load_and_gather(i, sv, dv, rv, sem):
        off = gbase + i * CHUNK
        pltpu.sync_copy(src_hbm.at[pl.ds(off, CHUNK)], sv)
        pltpu.sync_copy(dst_hbm.at[pl.ds(off, CHUNK)], dv)
        pltpu.async_copy(h_hbm.at[sv], rv, sem)

    def finish_and_scatter(sv, dv, rv, sem):
        pltpu.make_async_copy(h_hbm.at[sv], rv, sem).wait()
        pltpu.sync_copy(rv, agg_sh.at[dv], add=True)

    load_and_gather(0, src_a, dst_a, rows_a, gsem_a)

    @pl.loop(0, NFULL - 2, step=2)
    def _(i):
        load_and_gather(i + 1, src_b, dst_b, rows_b, gsem_b)
        finish_and_scatter(src_a, dst_a, rows_a, gsem_a)
        load_and_gather(i + 2, src_a, dst_a, rows_a, gsem_a)
        finish_and_scatter(src_b, dst_b, rows_b, gsem_b)

    load_and_gather(NFULL - 1, src_b, dst_b, rows_b, gsem_b)
    finish_and_scatter(src_a, dst_a, rows_a, gsem_a)
    toff = gbase + NFULL * CHUNK
    pltpu.sync_copy(src_hbm.at[pl.ds(toff, TAIL)], src_t)
    pltpu.sync_copy(dst_hbm.at[pl.ds(toff, TAIL)], dst_t)
    pltpu.async_copy(h_hbm.at[src_t], rows_t, gsem_a)
    finish_and_scatter(src_b, dst_b, rows_b, gsem_b)
    pltpu.make_async_copy(h_hbm.at[src_t], rows_t, gsem_a).wait()
    pltpu.sync_copy(rows_t, agg_sh.at[dst_t], add=True)
    plsc.subcore_barrier()

    _per_sub_chunks(
        s, lambda off: pltpu.sync_copy(agg_sh.at[pl.ds(off, ZCH)],
                                       agg_out.at[c, pl.ds(off, ZCH)]))


_sc_agg = pl.kernel(
    _sc_agg_body,
    out_type=(jax.ShapeDtypeStruct((NC, N, D), jnp.float32),),
    mesh=_MESH,
    scratch_types=[
        pltpu.VMEM((CHUNK,), jnp.int32),
        pltpu.VMEM((CHUNK,), jnp.int32),
        pltpu.VMEM((CHUNK,), jnp.int32),
        pltpu.VMEM((CHUNK,), jnp.int32),
        pltpu.VMEM((TAIL,), jnp.int32),
        pltpu.VMEM((TAIL,), jnp.int32),
        pltpu.VMEM((CHUNK, D), jnp.float32),
        pltpu.VMEM((CHUNK, D), jnp.float32),
        pltpu.VMEM((TAIL, D), jnp.float32),
        pltpu.VMEM((ZCH, D), jnp.float32),
        pltpu.VMEM_SHARED((N, D), jnp.float32),
        pltpu.SemaphoreType.DMA,
        pltpu.SemaphoreType.DMA,
    ])


def _sc_deg_body(dst_hbm, deg_out, dst_a, dst_b, dst_t, ones_v, zrow_v,
                 deg_sh, isem_a, isem_b):
    c = lax.axis_index("core")
    s = lax.axis_index("subcore")
    gbase = (c * NS + s) * EW

    _fill_vmem_2d(zrow_v, ZCH, D, 0.0)
    _per_sub_chunks(
        s, lambda off: pltpu.sync_copy(zrow_v, deg_sh.at[pl.ds(off, ZCH)]))
    _fill_vmem_2d(ones_v, CHUNK, D, 1.0)
    plsc.subcore_barrier()

    def load(i, dv, sem):
        off = gbase + i * CHUNK
        pltpu.async_copy(dst_hbm.at[pl.ds(off, CHUNK)], dv, sem)

    def scatter(dv, sem):
        pltpu.make_async_copy(dst_hbm.at[pl.ds(0, CHUNK)], dv, sem).wait()
        pltpu.sync_copy(ones_v, deg_sh.at[dv], add=True)

    load(0, dst_a, isem_a)

    @pl.loop(0, NFULL - 2, step=2)
    def _(i):
        load(i + 1, dst_b, isem_b)
        scatter(dst_a, isem_a)
        load(i + 2, dst_a, isem_a)
        scatter(dst_b, isem_b)

    load(NFULL - 1, dst_b, isem_b)
    scatter(dst_a, isem_a)
    toff = gbase + NFULL * CHUNK
    pltpu.sync_copy(dst_hbm.at[pl.ds(toff, TAIL)], dst_t)
    scatter(dst_b, isem_b)
    pltpu.sync_copy(ones_v.at[pl.ds(0, TAIL)], deg_sh.at[dst_t], add=True)
    plsc.subcore_barrier()

    _per_sub_chunks(
        s, lambda off: pltpu.sync_copy(deg_sh.at[pl.ds(off, ZCH)],
                                       deg_out.at[c, pl.ds(off, ZCH)]))


_sc_deg = pl.kernel(
    _sc_deg_body,
    out_type=(jax.ShapeDtypeStruct((NC, N, D), jnp.float32),),
    mesh=_MESH,
    scratch_types=[
        pltpu.VMEM((CHUNK,), jnp.int32),
        pltpu.VMEM((CHUNK,), jnp.int32),
        pltpu.VMEM((TAIL,), jnp.int32),
        pltpu.VMEM((CHUNK, D), jnp.float32),
        pltpu.VMEM((ZCH, D), jnp.float32),
        pltpu.VMEM_SHARED((N, D), jnp.float32),
        pltpu.SemaphoreType.DMA,
        pltpu.SemaphoreType.DMA,
    ])


BR = 1000


def _tc_layer0_body(h, a0, a1, d0, d1, ws, wn, b, g, be, o):
    deg = jnp.maximum(d0[:, :1] + d1[:, :1], 1.0)
    agg = (a0[...] + a1[...]) / deg
    z = (jnp.dot(h[...], ws[...], preferred_element_type=jnp.float32)
         + jnp.dot(agg, wn[...], preferred_element_type=jnp.float32)
         + b[...])
    mu = jnp.mean(z, axis=-1, keepdims=True)
    var = jnp.mean((z - mu) ** 2, axis=-1, keepdims=True)
    y = (z - mu) / jnp.sqrt(var + 1e-5) * g[...] + be[...]
    o[...] = jnp.maximum(y, 0.0)


def _tc_layer1_body(h, a0, a1, d0, d1, ws, wn, b, g, be, wl, bl, o):
    deg = jnp.maximum(d0[:, :1] + d1[:, :1], 1.0)
    agg = (a0[...] + a1[...]) / deg
    z = (jnp.dot(h[...], ws[...], preferred_element_type=jnp.float32)
         + jnp.dot(agg, wn[...], preferred_element_type=jnp.float32)
         + b[...])
    mu = jnp.mean(z, axis=-1, keepdims=True)
    var = jnp.mean((z - mu) ** 2, axis=-1, keepdims=True)
    y = (z - mu) / jnp.sqrt(var + 1e-5) * g[...] + be[...]
    y = jnp.maximum(y, 0.0)
    o[...] = jnp.dot(y, wl[...], preferred_element_type=jnp.float32) + bl[...]


def _row_spec(w):
    return pl.BlockSpec((BR, w), lambda i: (i, 0))


def _full_spec(r, c_):
    return pl.BlockSpec((r, c_), lambda i: (0, 0))


_tc_layer0 = pl.pallas_call(
    _tc_layer0_body,
    grid=(N // BR,),
    in_specs=[_row_spec(D), _row_spec(D), _row_spec(D),
              _row_spec(D), _row_spec(D),
              _full_spec(D, D), _full_spec(D, D),
              _full_spec(1, D), _full_spec(1, D), _full_spec(1, D)],
    out_specs=_row_spec(D),
    out_shape=jax.ShapeDtypeStruct((N, D), jnp.float32),
)

_tc_layer1 = pl.pallas_call(
    _tc_layer1_body,
    grid=(N // BR,),
    in_specs=[_row_spec(D), _row_spec(D), _row_spec(D),
              _row_spec(D), _row_spec(D),
              _full_spec(D, D), _full_spec(D, D),
              _full_spec(1, D), _full_spec(1, D), _full_spec(1, D),
              _full_spec(D, C), _full_spec(1, C)],
    out_specs=pl.BlockSpec((BR, C), lambda i: (i, 0)),
    out_shape=jax.ShapeDtypeStruct((N, C), jnp.float32),
)


def kernel(features, edge_index, W_self0, b_self0, W_neigh0, W_self1,
           b_self1, W_neigh1, W_lin, b_lin, g0, be0, g1, be1):
    src, dst = edge_index[0], edge_index[1]
    (deg_p,) = _sc_deg(dst)
    (agg_p,) = _sc_agg(features, src, dst)
    h1 = _tc_layer0(features, agg_p[0], agg_p[1], deg_p[0], deg_p[1],
                    W_self0, W_neigh0, b_self0.reshape(1, D),
                    g0.reshape(1, D), be0.reshape(1, D))
    (agg_p1,) = _sc_agg(h1, src, dst)
    out = _tc_layer1(h1, agg_p1[0], agg_p1[1], deg_p[0], deg_p[1],
                     W_self1, W_neigh1, b_self1.reshape(1, D),
                     g1.reshape(1, D), be1.reshape(1, D),
                     W_lin, b_lin.reshape(1, C))
    return out


# speedup vs baseline: 8.8245x; 1.0257x over previous
import jax
import jax.numpy as jnp
from jax import lax
from jax.experimental import pallas as pl
from jax.experimental.pallas import tpu as pltpu
from jax.experimental.pallas import tpu_sc as plsc

N = 10000
E = 320000
D = 128
C = 64

NC = 2
NS = 16
NW = NC * NS
EW = E // NW
CHUNK = 128
NFULL = EW // CHUNK
TAIL = EW - NFULL * CHUNK
SUB_STRIDE = 640
ZCH = 80

_MESH = plsc.VectorSubcoreMesh(core_axis_name="core",
                               subcore_axis_name="subcore",
                               num_cores=NC, num_subcores=NS)


def _fill_vmem_2d(ref, nrows, ncols, value):
    @pl.loop(0, nrows)
    def _(i):
        @pl.loop(0, ncols, step=16)
        def _(j):
            ref.at[pl.ds(i, 1), pl.ds(j, 16)][...] = jnp.full(
                (1, 16), value, jnp.float32)


def _per_sub_chunks(s, fn):
    for k in range(SUB_STRIDE // ZCH):
        off = s * SUB_STRIDE + k * ZCH

        @pl.when(off < N)
        def _():
            fn(off)


def _sc_agg_body(h_hbm, src_hbm, dst_hbm, agg_out, src_a, dst_a, src_b,
                 dst_b, src_t, dst_t, rows_a, rows_b, rows_t, zrow_v,
                 agg_sh, gsem_a, gsem_b, isem_a, isem_b):
    c = lax.axis_index("core")
    s = lax.axis_index("subcore")
    gbase = (c * NS + s) * EW

    _fill_vmem_2d(zrow_v, ZCH, D, 0.0)
    _per_sub_chunks(
        s, lambda off: pltpu.sync_copy(zrow_v, agg_sh.at[pl.ds(off, ZCH)]))
    plsc.subcore_barrier()

    def idxload(i, sv, dv, isem):
        off = gbase + i * CHUNK
        pltpu.async_copy(src_hbm.at[pl.ds(off, CHUNK)], sv, isem)
        pltpu.async_copy(dst_hbm.at[pl.ds(off, CHUNK)], dv, isem)

    def idxwait(sv, dv, isem):
        pltpu.make_async_copy(src_hbm.at[pl.ds(0, CHUNK)], sv, isem).wait()
        pltpu.make_async_copy(dst_hbm.at[pl.ds(0, CHUNK)], dv, isem).wait()

    def gather(sv, rv, gsem):
        pltpu.async_copy(h_hbm.at[sv], rv, gsem)

    def finish_and_scatter(sv, dv, rv, gsem):
        pltpu.make_async_copy(h_hbm.at[sv], rv, gsem).wait()
        pltpu.sync_copy(rv, agg_sh.at[dv], add=True)

    idxload(0, src_a, dst_a, isem_a)
    idxload(1, src_b, dst_b, isem_b)
    idxwait(src_a, dst_a, isem_a)
    gather(src_a, rows_a, gsem_a)

    @pl.loop(0, NFULL, step=2)
    def _(i):
        idxwait(src_b, dst_b, isem_b)
        gather(src_b, rows_b, gsem_b)
        finish_and_scatter(src_a, dst_a, rows_a, gsem_a)
        idxload(i + 2, src_a, dst_a, isem_a)
        finish_and_scatter(src_b, dst_b, rows_b, gsem_b)
        idxload(i + 3, src_b, dst_b, isem_b)
        idxwait(src_a, dst_a, isem_a)
        gather(src_a, rows_a, gsem_a)

    pltpu.make_async_copy(h_hbm.at[src_a], rows_a, gsem_a).wait()
    idxwait(src_b, dst_b, isem_b)
    toff = gbase + NFULL * CHUNK
    pltpu.sync_copy(src_hbm.at[pl.ds(toff, TAIL)], src_t)
    pltpu.sync_copy(dst_hbm.at[pl.ds(toff, TAIL)], dst_t)
    pltpu.sync_copy(h_hbm.at[src_t], rows_t)
    pltpu.sync_copy(rows_t, agg_sh.at[dst_t], add=True)
    plsc.subcore_barrier()

    _per_sub_chunks(
        s, lambda off: pltpu.sync_copy(agg_sh.at[pl.ds(off, ZCH)],
                                       agg_out.at[c, pl.ds(off, ZCH)]))


_sc_agg = pl.kernel(
    _sc_agg_body,
    out_type=(jax.ShapeDtypeStruct((NC, N, D), jnp.float32),),
    mesh=_MESH,
    scratch_types=[
        pltpu.VMEM((CHUNK,), jnp.int32),
        pltpu.VMEM((CHUNK,), jnp.int32),
        pltpu.VMEM((CHUNK,), jnp.int32),
        pltpu.VMEM((CHUNK,), jnp.int32),
        pltpu.VMEM((TAIL,), jnp.int32),
        pltpu.VMEM((TAIL,), jnp.int32),
        pltpu.VMEM((CHUNK, D), jnp.float32),
        pltpu.VMEM((CHUNK, D), jnp.float32),
        pltpu.VMEM((TAIL, D), jnp.float32),
        pltpu.VMEM((ZCH, D), jnp.float32),
        pltpu.VMEM_SHARED((N, D), jnp.float32),
        pltpu.SemaphoreType.DMA,
        pltpu.SemaphoreType.DMA,
        pltpu.SemaphoreType.DMA,
        pltpu.SemaphoreType.DMA,
    ])


def _sc_deg_body(dst_hbm, deg_out, dst_a, dst_b, dst_t, ones_v, zrow_v,
                 deg_sh, isem_a, isem_b):
    c = lax.axis_index("core")
    s = lax.axis_index("subcore")
    gbase = (c * NS + s) * EW

    _fill_vmem_2d(zrow_v, ZCH, D, 0.0)
    _per_sub_chunks(
        s, lambda off: pltpu.sync_copy(zrow_v, deg_sh.at[pl.ds(off, ZCH)]))
    _fill_vmem_2d(ones_v, CHUNK, D, 1.0)
    plsc.subcore_barrier()

    def load(i, dv, sem):
        off = gbase + i * CHUNK
        pltpu.async_copy(dst_hbm.at[pl.ds(off, CHUNK)], dv, sem)

    def scatter(dv, sem):
        pltpu.make_async_copy(dst_hbm.at[pl.ds(0, CHUNK)], dv, sem).wait()
        pltpu.sync_copy(ones_v, deg_sh.at[dv], add=True)

    load(0, dst_a, isem_a)

    @pl.loop(0, NFULL - 2, step=2)
    def _(i):
        load(i + 1, dst_b, isem_b)
        scatter(dst_a, isem_a)
        load(i + 2, dst_a, isem_a)
        scatter(dst_b, isem_b)

    load(NFULL - 1, dst_b, isem_b)
    scatter(dst_a, isem_a)
    toff = gbase + NFULL * CHUNK
    pltpu.sync_copy(dst_hbm.at[pl.ds(toff, TAIL)], dst_t)
    scatter(dst_b, isem_b)
    pltpu.sync_copy(ones_v.at[pl.ds(0, TAIL)], deg_sh.at[dst_t], add=True)
    plsc.subcore_barrier()

    _per_sub_chunks(
        s, lambda off: pltpu.sync_copy(deg_sh.at[pl.ds(off, ZCH)],
                                       deg_out.at[c, pl.ds(off, ZCH)]))


_sc_deg = pl.kernel(
    _sc_deg_body,
    out_type=(jax.ShapeDtypeStruct((NC, N, D), jnp.float32),),
    mesh=_MESH,
    scratch_types=[
        pltpu.VMEM((CHUNK,), jnp.int32),
        pltpu.VMEM((CHUNK,), jnp.int32),
        pltpu.VMEM((TAIL,), jnp.int32),
        pltpu.VMEM((CHUNK, D), jnp.float32),
        pltpu.VMEM((ZCH, D), jnp.float32),
        pltpu.VMEM_SHARED((N, D), jnp.float32),
        pltpu.SemaphoreType.DMA,
        pltpu.SemaphoreType.DMA,
    ])


BR = 1000


def _tc_layer0_body(h, a0, a1, d0, d1, ws, wn, b, g, be, o):
    deg = jnp.maximum(d0[:, :1] + d1[:, :1], 1.0)
    agg = (a0[...] + a1[...]) / deg
    z = (jnp.dot(h[...], ws[...], preferred_element_type=jnp.float32)
         + jnp.dot(agg, wn[...], preferred_element_type=jnp.float32)
         + b[...])
    mu = jnp.mean(z, axis=-1, keepdims=True)
    var = jnp.mean((z - mu) ** 2, axis=-1, keepdims=True)
    y = (z - mu) / jnp.sqrt(var + 1e-5) * g[...] + be[...]
    o[...] = jnp.maximum(y, 0.0)


def _tc_layer1_body(h, a0, a1, d0, d1, ws, wn, b, g, be, wl, bl, o):
    deg = jnp.maximum(d0[:, :1] + d1[:, :1], 1.0)
    agg = (a0[...] + a1[...]) / deg
    z = (jnp.dot(h[...], ws[...], preferred_element_type=jnp.float32)
         + jnp.dot(agg, wn[...], preferred_element_type=jnp.float32)
         + b[...])
    mu = jnp.mean(z, axis=-1, keepdims=True)
    var = jnp.mean((z - mu) ** 2, axis=-1, keepdims=True)
    y = (z - mu) / jnp.sqrt(var + 1e-5) * g[...] + be[...]
    y = jnp.maximum(y, 0.0)
    o[...] = jnp.dot(y, wl[...], preferred_element_type=jnp.float32) + bl[...]


def _row_spec(w):
    return pl.BlockSpec((BR, w), lambda i: (i, 0))


def _full_spec(r, c_):
    return pl.BlockSpec((r, c_), lambda i: (0, 0))


_tc_layer0 = pl.pallas_call(
    _tc_layer0_body,
    grid=(N // BR,),
    in_specs=[_row_spec(D), _row_spec(D), _row_spec(D),
              _row_spec(D), _row_spec(D),
              _full_spec(D, D), _full_spec(D, D),
              _full_spec(1, D), _full_spec(1, D), _full_spec(1, D)],
    out_specs=_row_spec(D),
    out_shape=jax.ShapeDtypeStruct((N, D), jnp.float32),
)

_tc_layer1 = pl.pallas_call(
    _tc_layer1_body,
    grid=(N // BR,),
    in_specs=[_row_spec(D), _row_spec(D), _row_spec(D),
              _row_spec(D), _row_spec(D),
              _full_spec(D, D), _full_spec(D, D),
              _full_spec(1, D), _full_spec(1, D), _full_spec(1, D),
              _full_spec(D, C), _full_spec(1, C)],
    out_specs=pl.BlockSpec((BR, C), lambda i: (i, 0)),
    out_shape=jax.ShapeDtypeStruct((N, C), jnp.float32),
)


def kernel(features, edge_index, W_self0, b_self0, W_neigh0, W_self1,
           b_self1, W_neigh1, W_lin, b_lin, g0, be0, g1, be1):
    src, dst = edge_index[0], edge_index[1]
    pad = jnp.zeros((2 * CHUNK,), jnp.int32)
    src_p = jnp.concatenate([src, pad])
    dst_p = jnp.concatenate([dst, pad])
    (deg_p,) = _sc_deg(dst)
    (agg_p,) = _sc_agg(features, src_p, dst_p)
    h1 = _tc_layer0(features, agg_p[0], agg_p[1], deg_p[0], deg_p[1],
                    W_self0, W_neigh0, b_self0.reshape(1, D),
                    g0.reshape(1, D), be0.reshape(1, D))
    (agg_p1,) = _sc_agg(h1, src_p, dst_p)
    out = _tc_layer1(h1, agg_p1[0], agg_p1[1], deg_p[0], deg_p[1],
                     W_self1, W_neigh1, b_self1.reshape(1, D),
                     g1.reshape(1, D), be1.reshape(1, D),
                     W_lin, b_lin.reshape(1, C))
    return out
